# Initial kernel scaffold; baseline (speedup 1.0000x reference)
import jax
import jax.numpy as jnp
from jax.experimental import pallas as pl


def kernel(input_ids, embedding, proj_w):
    raise NotImplementedError("write your pallas kernel here")


# speedup vs baseline: 1.6642x; 1.6642x over previous
import jax
import jax.numpy as jnp
from jax.experimental import pallas as pl

HIDDEN = 768
NUM_EMB = 32
VOCAB = 8192
V_BLK = 1024


def _kern(ids_ref, emb_ref, pw_ref, out_ref):
    table = jax.lax.dot_general(
        emb_ref[...], pw_ref[...],
        (((1,), (1,)), ((), ())),
        preferred_element_type=jnp.float32,
    )
    ids = ids_ref[0, :]
    s = ids.shape[0]
    onehot = (ids[:, None] == jax.lax.broadcasted_iota(jnp.int32, (s, NUM_EMB), 1)
              ).astype(jnp.float32)
    out_ref[0, :, :] = jnp.dot(onehot, table, preferred_element_type=jnp.float32)


def kernel(input_ids, embedding, proj_w):
    b, s = input_ids.shape
    return pl.pallas_call(
        _kern,
        grid=(VOCAB // V_BLK,),
        in_specs=[
            pl.BlockSpec((b, s), lambda j: (0, 0)),
            pl.BlockSpec((NUM_EMB, HIDDEN), lambda j: (0, 0)),
            pl.BlockSpec((V_BLK, HIDDEN), lambda j: (j, 0)),
        ],
        out_specs=pl.BlockSpec((b, s, V_BLK), lambda j: (0, 0, j)),
        out_shape=jax.ShapeDtypeStruct((b, s, VOCAB), jnp.float32),
    )(input_ids, embedding, proj_w)
